# Initial kernel scaffold; baseline (speedup 1.0000x reference)
import jax
import jax.numpy as jnp
from jax.experimental import pallas as pl


def kernel(x, edge_index_r0, edge_index_r1, edge_index_r2, W_r0, b_r0, W_r1, b_r1, W_r2, b_r2):
    raise NotImplementedError("write your pallas kernel here")


# speedup vs baseline: 7.3818x; 7.3818x over previous
import functools

import jax
import jax.numpy as jnp
from jax import lax
from jax.experimental import pallas as pl
from jax.experimental.pallas import tpu as pltpu
from jax.experimental.pallas import tpu_sc as plsc

N = 10000
D = 128
E = 320000
NC, NS, L = 2, 16, 16
NW = NC * NS

N_PAD = 10240
BLK = 128
NBLK_E = E // BLK
A_BLKS, A_EXTRA = NBLK_E // NW, NBLK_E % NW
EPT_MAX = (A_BLKS + 1) * BLK
ROWS_PT = N_PAD // NS
ZROWS = 128

_mesh = plsc.VectorSubcoreMesh(
    core_axis_name="c", subcore_axis_name="s", num_cores=NC, num_subcores=NS)


@functools.partial(
    pl.kernel,
    out_type=jax.ShapeDtypeStruct((6, NW, N_PAD), jnp.float32),
    mesh=_mesh,
    scratch_types=[
        pltpu.VMEM((N_PAD,), jnp.float32),
        pltpu.VMEM((EPT_MAX,), jnp.int32),
    ],
    compiler_params=pltpu.CompilerParams(needs_layout_passes=False),
)
def _deg_kernel(e0, e1, e2, out, deg_v, idx_v):
    cid = lax.axis_index("c")
    sid = lax.axis_index("s")
    wid = sid * NC + cid
    has_extra = wid < A_EXTRA
    start = (wid * A_BLKS + jnp.minimum(wid, A_EXTRA)) * BLK
    nvec8 = A_BLKS + jnp.where(has_extra, 1, 0)
    ones = jnp.ones((L,), jnp.float32)
    zeros = jnp.zeros((L,), jnp.float32)
    for a in range(6):
        which = a // 3
        er = (e0, e1, e2)[a % 3]

        def zbody(j, c):
            for u in range(8):
                deg_v[pl.ds((j * 8 + u) * L, L)] = zeros
            return c
        lax.fori_loop(0, N_PAD // (8 * L), zbody, 0)

        @pl.when(has_extra)
        def _():
            pltpu.sync_copy(er.at[pl.ds(which * E + start, EPT_MAX)], idx_v)

        @pl.when(jnp.logical_not(has_extra))
        def _():
            pltpu.sync_copy(er.at[pl.ds(which * E + start, A_BLKS * BLK)],
                            idx_v.at[pl.ds(0, A_BLKS * BLK)])

        def body(j, c):
            for u in range(8):
                iv = idx_v[pl.ds((j * 8 + u) * L, L)]
                plsc.addupdate_scatter(deg_v, [iv], ones)
            return c
        lax.fori_loop(0, nvec8, body, 0)

        pltpu.sync_copy(deg_v, out.at[a, wid])


def _norm_h_body(degs_ref, x_ref, h0_ref, h1_ref, h2_ref, nd_ref):
    deg = jnp.sum(degs_ref[...], axis=1)
    norm = jnp.where(deg > 0, lax.rsqrt(jnp.maximum(deg, 1e-12)), 0.0)
    x = x_ref[...]
    for r, h_ref in enumerate((h0_ref, h1_ref, h2_ref)):
        h_ref[...] = x * norm[r][:, None]
    nd_ref[...] = norm[3:6]


NBLK = 2048


def _norm_h(degs, x):
    grid = (pl.cdiv(N, NBLK),)
    return pl.pallas_call(
        _norm_h_body,
        grid=grid,
        in_specs=[
            pl.BlockSpec((6, NW, NBLK), lambda i: (0, 0, i)),
            pl.BlockSpec((NBLK, D), lambda i: (i, 0)),
        ],
        out_specs=[
            pl.BlockSpec((NBLK, D), lambda i: (i, 0)),
            pl.BlockSpec((NBLK, D), lambda i: (i, 0)),
            pl.BlockSpec((NBLK, D), lambda i: (i, 0)),
            pl.BlockSpec((3, NBLK), lambda i: (0, i)),
        ],
        out_shape=[
            jax.ShapeDtypeStruct((N, D), jnp.float32),
            jax.ShapeDtypeStruct((N, D), jnp.float32),
            jax.ShapeDtypeStruct((N, D), jnp.float32),
            jax.ShapeDtypeStruct((3, N), jnp.float32),
        ],
    )(degs, x)


@functools.partial(
    pl.kernel,
    out_type=jax.ShapeDtypeStruct((3, NC, N_PAD, D), jnp.float32),
    mesh=_mesh,
    scratch_types=[
        pltpu.VMEM_SHARED((N_PAD, D), jnp.float32),
        pltpu.VMEM((BLK,), jnp.int32),
        pltpu.VMEM((BLK,), jnp.int32),
        pltpu.VMEM((BLK, D), jnp.float32),
        pltpu.VMEM((ZROWS, D), jnp.float32),
        pltpu.SemaphoreType.DMA,
    ],
    compiler_params=pltpu.CompilerParams(needs_layout_passes=False),
)
def _agg_kernel(h0, h1, h2, e0, e1, e2, out, acc_sh, sidx, didx, rows, zbuf,
                sem):
    cid = lax.axis_index("c")
    sid = lax.axis_index("s")
    zeros = jnp.zeros((L,), jnp.float32)

    def zb(i, c):
        for u in range(D // L):
            zbuf[i, pl.ds(u * L, L)] = zeros
        return c
    lax.fori_loop(0, ZROWS, zb, 0)

    e_half = E // NC
    blks_per_core = e_half // BLK
    nblk = blks_per_core // NS + jnp.where(
        sid < blks_per_core % NS, 1, 0)

    for r in range(3):
        er = (e0, e1, e2)[r]
        hr = (h0, h1, h2)[r]

        for j in range(ROWS_PT // ZROWS):
            pltpu.sync_copy(
                zbuf, acc_sh.at[pl.ds(sid * ROWS_PT + j * ZROWS, ZROWS)])
        plsc.subcore_barrier()

        def ebody(k, c):
            off = cid * e_half + (sid + k * NS) * BLK
            pltpu.sync_copy(er.at[pl.ds(off, BLK)], sidx)
            pltpu.sync_copy(er.at[pl.ds(E + off, BLK)], didx)
            pltpu.async_copy(hr.at[sidx], rows, sem).wait()
            pltpu.sync_copy(rows, acc_sh.at[didx], add=True)
            return c
        lax.fori_loop(0, nblk, ebody, 0)
        plsc.subcore_barrier()

        pltpu.sync_copy(acc_sh.at[pl.ds(sid * ROWS_PT, ROWS_PT)],
                        out.at[r, cid, pl.ds(sid * ROWS_PT, ROWS_PT)])


def _final_body(aggp_ref, nd_ref, W_ref, bm_ref, out_ref):
    nd = nd_ref[...]
    acc = bm_ref[...] * jnp.ones((aggp_ref.shape[2], 1), jnp.float32)
    for r in range(3):
        s = (aggp_ref[r, 0] + aggp_ref[r, 1]) * nd[r][:, None]
        acc = acc + (1.0 / 3.0) * jnp.dot(
            s, W_ref[r], preferred_element_type=jnp.float32)
    out_ref[...] = acc


def _final(aggp, nd, Ws, bm):
    grid = (pl.cdiv(N, NBLK),)
    return pl.pallas_call(
        _final_body,
        grid=grid,
        in_specs=[
            pl.BlockSpec((3, NC, NBLK, D), lambda i: (0, 0, i, 0)),
            pl.BlockSpec((3, NBLK), lambda i: (0, i)),
            pl.BlockSpec((3, D, D), lambda i: (0, 0, 0)),
            pl.BlockSpec((1, D), lambda i: (0, 0)),
        ],
        out_specs=pl.BlockSpec((NBLK, D), lambda i: (i, 0)),
        out_shape=jax.ShapeDtypeStruct((N, D), jnp.float32),
    )(aggp, nd, Ws, bm)


def kernel(x, edge_index_r0, edge_index_r1, edge_index_r2,
           W_r0, b_r0, W_r1, b_r1, W_r2, b_r2):
    e0 = edge_index_r0.reshape(2 * E)
    e1 = edge_index_r1.reshape(2 * E)
    e2 = edge_index_r2.reshape(2 * E)
    degs = _deg_kernel(e0, e1, e2)
    h0, h1, h2, nd = _norm_h(degs, x)
    aggp = _agg_kernel(h0, h1, h2, e0, e1, e2)
    Ws = jnp.stack([W_r0, W_r1, W_r2])
    bm = ((b_r0 + b_r1 + b_r2) / 3.0).reshape(1, D)
    return _final(aggp, nd, Ws, bm)
